# Initial kernel scaffold; baseline (speedup 1.0000x reference)
import jax
import jax.numpy as jnp
from jax.experimental import pallas as pl


def kernel(indices):
    raise NotImplementedError("write your pallas kernel here")


# speedup vs baseline: 1.0555x; 1.0555x over previous
import jax
import jax.numpy as jnp
from jax import lax
from jax.experimental import pallas as pl
from jax.experimental.pallas import tpu as pltpu
from jax.experimental.pallas import tpu_sc as plsc

NUM_CLASSES = 1000
BATCH = 16384
NC, NS, L = 2, 16, 16
NW = NC * NS
ROWS_W = BATCH // NW
CHUNK = 16
NITER = ROWS_W // CHUNK
CHUNK_ELEMS = CHUNK * NUM_CLASSES


def _sc_body(idx_hbm, out_hbm, idx_v, buf0, buf1, sem0, sem1):
    wid = lax.axis_index("s") * NC + lax.axis_index("c")
    row0 = wid * ROWS_W
    pltpu.sync_copy(idx_hbm.at[pl.ds(row0, ROWS_W)], idx_v)

    bufs = (buf0, buf1)
    sems = (sem0, sem1)
    zeros16 = jnp.zeros((L,), jnp.float32)
    ones16 = jnp.ones((L,), jnp.float32)
    lane_base = lax.iota(jnp.int32, L) * NUM_CLASSES

    def zbody(k, _):
        off = k * (L * 4)
        for u in range(4):
            bufs[0][pl.ds(off + u * L, L)] = zeros16
            bufs[1][pl.ds(off + u * L, L)] = zeros16
        return 0

    lax.fori_loop(0, CHUNK_ELEMS // (L * 4), zbody, 0)

    out_base = row0 * NUM_CLASSES

    def body(j, _):
        for b in range(2):
            i = j * 2 + b
            buf = bufs[b]

            @pl.when(j >= 1)
            def _wait_and_clear():
                pltpu.make_async_copy(
                    buf, out_hbm.at[pl.ds(out_base, CHUNK_ELEMS)], sems[b]
                ).wait()
                old_idx = idx_v[pl.ds((i - 2) * L, L)]
                plsc.store_scatter(buf, [lane_base + old_idx], zeros16)

            new_idx = idx_v[pl.ds(i * L, L)]
            plsc.store_scatter(buf, [lane_base + new_idx], ones16)
            pltpu.async_copy(
                buf,
                out_hbm.at[pl.ds(out_base + i * CHUNK_ELEMS, CHUNK_ELEMS)],
                sems[b],
            )
        return 0

    lax.fori_loop(0, NITER // 2, body, 0)

    for b in range(2):
        pltpu.make_async_copy(
            bufs[b], out_hbm.at[pl.ds(out_base, CHUNK_ELEMS)], sems[b]
        ).wait()


def kernel(indices):
    k = pl.kernel(
        _sc_body,
        out_type=jax.ShapeDtypeStruct((BATCH * NUM_CLASSES,), jnp.float32),
        mesh=plsc.VectorSubcoreMesh(
            core_axis_name="c", subcore_axis_name="s",
            num_cores=NC, num_subcores=NS,
        ),
        scratch_types=[
            pltpu.VMEM((ROWS_W,), jnp.int32),
            pltpu.VMEM((CHUNK_ELEMS,), jnp.float32),
            pltpu.VMEM((CHUNK_ELEMS,), jnp.float32),
            pltpu.SemaphoreType.DMA,
            pltpu.SemaphoreType.DMA,
        ],
        compiler_params=pltpu.CompilerParams(needs_layout_passes=False),
    )
    flat = k(indices.astype(jnp.int32))
    return flat.reshape(BATCH, NUM_CLASSES)


# speedup vs baseline: 1.7026x; 1.6131x over previous
import jax
import jax.numpy as jnp
from jax import lax
from jax.experimental import pallas as pl
from jax.experimental.pallas import tpu as pltpu
from jax.experimental.pallas import tpu_sc as plsc

NUM_CLASSES = 1000
BATCH = 16384
NC, NS, L = 2, 16, 16
NW = NC * NS
ROWS_W = BATCH // NW
CHUNK = 16
NITER = ROWS_W // CHUNK
_COL_OFFS = tuple(range(0, NUM_CLASSES - L + 1, L)) + (NUM_CLASSES - L,)


def _sc_body(idx_hbm, out_hbm, idx_v, buf0, buf1, sem0, sem1):
    wid = lax.axis_index("s") * NC + lax.axis_index("c")
    row0 = wid * ROWS_W
    pltpu.sync_copy(idx_hbm.at[pl.ds(row0, ROWS_W)], idx_v)

    bufs = (buf0, buf1)
    sems = (sem0, sem1)
    zeros16 = jnp.zeros((L,), jnp.float32)
    ones16 = jnp.ones((L,), jnp.float32)
    lane_iota = lax.iota(jnp.int32, L)

    def zbody(r, _):
        for c in _COL_OFFS:
            bufs[0][r, pl.ds(c, L)] = zeros16
            bufs[1][r, pl.ds(c, L)] = zeros16
        return 0

    lax.fori_loop(0, CHUNK, zbody, 0)

    def body(j, _):
        for b in range(2):
            i = j * 2 + b
            buf = bufs[b]

            @pl.when(j >= 1)
            def _wait_and_clear():
                pltpu.make_async_copy(
                    buf, out_hbm.at[pl.ds(row0, CHUNK)], sems[b]
                ).wait()
                old_idx = idx_v[pl.ds((i - 2) * L, L)]
                plsc.store_scatter(buf, [lane_iota, old_idx], zeros16)

            new_idx = idx_v[pl.ds(i * L, L)]
            plsc.store_scatter(buf, [lane_iota, new_idx], ones16)
            pltpu.async_copy(
                buf, out_hbm.at[pl.ds(row0 + i * CHUNK, CHUNK)], sems[b]
            )
        return 0

    lax.fori_loop(0, NITER // 2, body, 0)

    for b in range(2):
        pltpu.make_async_copy(
            bufs[b], out_hbm.at[pl.ds(row0, CHUNK)], sems[b]
        ).wait()


def kernel(indices):
    k = pl.kernel(
        _sc_body,
        out_type=jax.ShapeDtypeStruct((BATCH, NUM_CLASSES), jnp.float32),
        mesh=plsc.VectorSubcoreMesh(
            core_axis_name="c", subcore_axis_name="s",
            num_cores=NC, num_subcores=NS,
        ),
        scratch_types=[
            pltpu.VMEM((ROWS_W,), jnp.int32),
            pltpu.VMEM((CHUNK, NUM_CLASSES), jnp.float32),
            pltpu.VMEM((CHUNK, NUM_CLASSES), jnp.float32),
            pltpu.SemaphoreType.DMA,
            pltpu.SemaphoreType.DMA,
        ],
        compiler_params=pltpu.CompilerParams(needs_layout_passes=False),
    )
    return k(indices.astype(jnp.int32))


# speedup vs baseline: 1.7197x; 1.0101x over previous
import jax
import jax.numpy as jnp
from jax import lax
from jax.experimental import pallas as pl
from jax.experimental.pallas import tpu as pltpu
from jax.experimental.pallas import tpu_sc as plsc

NUM_CLASSES = 1000
BATCH = 16384
NC, NS, L = 2, 16, 16
NW = NC * NS
ROWS_W = BATCH // NW
CHUNK = 16
NITER = ROWS_W // CHUNK
_COL_OFFS = tuple(range(0, NUM_CLASSES - L + 1, L)) + (NUM_CLASSES - L,)


def _sc_body(idx_hbm, out_hbm, idx_v, buf0, buf1, sem0, sem1):
    wid = lax.axis_index("s") * NC + lax.axis_index("c")
    row0 = wid * ROWS_W
    pltpu.sync_copy(idx_hbm.at[pl.ds(row0, ROWS_W)], idx_v)

    bufs = (buf0, buf1)
    sems = (sem0, sem1)
    zeros16 = jnp.zeros((L,), jnp.float32)
    ones16 = jnp.ones((L,), jnp.float32)
    lane_iota = lax.iota(jnp.int32, L)

    def zbody(r, _):
        for c in _COL_OFFS:
            bufs[0][r, pl.ds(c, L)] = zeros16
            bufs[1][r, pl.ds(c, L)] = zeros16
        return 0

    lax.fori_loop(0, CHUNK, zbody, 0)

    def body(j, _):
        for b in range(2):
            i = j * 2 + b
            buf = bufs[b]

            @pl.when(j >= 1)
            def _wait_and_clear():
                pltpu.make_async_copy(
                    buf, out_hbm.at[pl.ds(row0, CHUNK)], sems[b]
                ).wait()
                old_idx = idx_v[pl.ds((i - 2) * L, L)]
                plsc.store_scatter(buf, [lane_iota, old_idx], zeros16)

            new_idx = idx_v[pl.ds(i * L, L)]
            plsc.store_scatter(buf, [lane_iota, new_idx], ones16)
            pltpu.async_copy(
                buf, out_hbm.at[pl.ds(row0 + i * CHUNK, CHUNK)], sems[b]
            )
        return 0

    lax.fori_loop(0, NITER // 2, body, 0)

    for b in range(2):
        pltpu.make_async_copy(
            bufs[b], out_hbm.at[pl.ds(row0, CHUNK)], sems[b]
        ).wait()


def kernel(indices):
    k = pl.kernel(
        _sc_body,
        out_type=jax.ShapeDtypeStruct((BATCH, NUM_CLASSES), jnp.float32),
        mesh=plsc.VectorSubcoreMesh(
            core_axis_name="c", subcore_axis_name="s",
            num_cores=NC, num_subcores=NS,
        ),
        scratch_types=[
            pltpu.VMEM((ROWS_W,), jnp.int32),
            pltpu.VMEM((CHUNK, NUM_CLASSES), jnp.float32),
            pltpu.VMEM((CHUNK, NUM_CLASSES), jnp.float32),
            pltpu.SemaphoreType.DMA,
            pltpu.SemaphoreType.DMA,
        ],
        compiler_params=pltpu.CompilerParams(
            needs_layout_passes=False, use_tc_tiling_on_sc=True
        ),
    )
    return k(indices.astype(jnp.int32))


# speedup vs baseline: 3.7194x; 2.1628x over previous
import jax
import jax.numpy as jnp
from jax import lax
from jax.experimental import pallas as pl
from jax.experimental.pallas import tpu as pltpu
from jax.experimental.pallas import tpu_sc as plsc

NUM_CLASSES = 1000
BATCH = 16384
NC, NS, L = 2, 16, 16
NW = NC * NS
COLS_W = BATCH // NW
R = 40
NCH = NUM_CLASSES // R
NVREG = COLS_W // L


def _sc_body(idx_hbm, out_hbm, idx_v, buf0, buf1, sem0, sem1):
    wid = lax.axis_index("s") * NC + lax.axis_index("c")
    c0 = wid * COLS_W
    pltpu.sync_copy(idx_hbm.at[pl.ds(c0, COLS_W)], idx_v)

    bufs = (buf0, buf1)
    sems = (sem0, sem1)
    zeros16 = jnp.zeros((L,), jnp.float32)
    ones16 = jnp.ones((L,), jnp.float32)
    lane_iota = lax.iota(jnp.int32, L)

    def zbody(r, _):
        for c in range(0, COLS_W, L):
            bufs[0][r, pl.ds(c, L)] = zeros16
            bufs[1][r, pl.ds(c, L)] = zeros16
        return 0

    lax.fori_loop(0, R, zbody, 0)

    def process_chunk(i, b, first):
        buf = bufs[b]
        rbase_new = i * R
        rbase_old = rbase_new - 2 * R

        if not first:
            pltpu.make_async_copy(
                buf, out_hbm.at[pl.ds(0, R), pl.ds(c0, COLS_W)], sems[b]
            ).wait()

        for v in range(NVREG):
            idxv = idx_v[pl.ds(v * L, L)]
            col = lane_iota + (v * L)
            old_row = idxv - rbase_old
            old_mask = (old_row >= 0) & (old_row < R)
            old_row = jnp.where(old_mask, old_row, 0)
            plsc.store_scatter(buf, [old_row, col], zeros16, mask=old_mask)
            new_row = idxv - rbase_new
            new_mask = (new_row >= 0) & (new_row < R)
            new_row = jnp.where(new_mask, new_row, 0)
            plsc.store_scatter(buf, [new_row, col], ones16, mask=new_mask)

        pltpu.async_copy(
            buf, out_hbm.at[pl.ds(rbase_new, R), pl.ds(c0, COLS_W)], sems[b]
        )

    def body(j, _):
        for b in range(2):
            i = j * 2 + b

            @pl.when(j == 0)
            def _first():
                process_chunk(i, b, True)

            @pl.when(j > 0)
            def _steady():
                process_chunk(i, b, False)

        return 0

    lax.fori_loop(0, NCH // 2, body, 0)
    process_chunk(NCH - 1, 0, False)
    pltpu.make_async_copy(
        bufs[1], out_hbm.at[pl.ds(0, R), pl.ds(c0, COLS_W)], sems[1]
    ).wait()
    pltpu.make_async_copy(
        bufs[0], out_hbm.at[pl.ds(0, R), pl.ds(c0, COLS_W)], sems[0]
    ).wait()


def kernel(indices):
    k = pl.kernel(
        _sc_body,
        out_type=jax.ShapeDtypeStruct((NUM_CLASSES, BATCH), jnp.float32),
        mesh=plsc.VectorSubcoreMesh(
            core_axis_name="c", subcore_axis_name="s",
            num_cores=NC, num_subcores=NS,
        ),
        scratch_types=[
            pltpu.VMEM((COLS_W,), jnp.int32),
            pltpu.VMEM((R, COLS_W), jnp.float32),
            pltpu.VMEM((R, COLS_W), jnp.float32),
            pltpu.SemaphoreType.DMA,
            pltpu.SemaphoreType.DMA,
        ],
        compiler_params=pltpu.CompilerParams(
            needs_layout_passes=False, use_tc_tiling_on_sc=True
        ),
    )
    return k(indices.astype(jnp.int32)).T


# speedup vs baseline: 3.7374x; 1.0049x over previous
import jax
import jax.numpy as jnp
from jax import lax
from jax.experimental import pallas as pl
from jax.experimental.pallas import tpu as pltpu
from jax.experimental.pallas import tpu_sc as plsc

NUM_CLASSES = 1000
BATCH = 16384
NC, NS, L = 2, 16, 16
NW = NC * NS
COLS_W = BATCH // NW
R = 64
NCH = -(-NUM_CLASSES // R)
R_LAST = NUM_CLASSES - (NCH - 1) * R
NVREG = COLS_W // L


def _sc_body(idx_hbm, out_hbm, idx_v, buf0, buf1, sem0, sem1):
    wid = lax.axis_index("s") * NC + lax.axis_index("c")
    c0 = wid * COLS_W
    pltpu.sync_copy(idx_hbm.at[pl.ds(c0, COLS_W)], idx_v)

    bufs = (buf0, buf1)
    sems = (sem0, sem1)
    zeros16 = jnp.zeros((L,), jnp.float32)
    ones16 = jnp.ones((L,), jnp.float32)
    lane_iota = lax.iota(jnp.int32, L)

    def zero_buf(buf):
        def zbody(r, _):
            for c in range(0, COLS_W, L):
                buf[r, pl.ds(c, L)] = zeros16
            return 0

        lax.fori_loop(0, R, zbody, 0)

    def in_range(row, n):
        return plsc.bitcast(row, jnp.uint32) < jnp.uint32(n)

    def process_chunk(i, b, rows_new, first):
        buf = bufs[b]
        rbase_new = i * R

        if not first:
            pltpu.make_async_copy(
                buf, out_hbm.at[pl.ds(0, R), pl.ds(c0, COLS_W)], sems[b]
            ).wait()

        for v in range(NVREG):
            idxv = idx_v[pl.ds(v * L, L)]
            col = lane_iota + (v * L)
            new_row = idxv - rbase_new
            old_row = new_row + 2 * R
            plsc.store_scatter(
                buf, [old_row, col], zeros16, mask=in_range(old_row, R)
            )
            plsc.store_scatter(
                buf, [new_row, col], ones16, mask=in_range(new_row, rows_new)
            )

        if rows_new == R:
            src, dst = buf, out_hbm.at[pl.ds(rbase_new, R), pl.ds(c0, COLS_W)]
        else:
            src = buf.at[pl.ds(0, rows_new)]
            dst = out_hbm.at[pl.ds(rbase_new, rows_new), pl.ds(c0, COLS_W)]
        pltpu.async_copy(src, dst, sems[b])

    zero_buf(buf0)
    process_chunk(0, 0, R, True)
    zero_buf(buf1)
    process_chunk(1, 1, R, True)

    def body(j, _):
        for b in range(2):
            process_chunk(j * 2 + b, b, R, False)
        return 0

    lax.fori_loop(1, (NCH - 2) // 2, body, 0)
    process_chunk(NCH - 2, 0, R, False)
    process_chunk(NCH - 1, 1, R_LAST, False)

    pltpu.make_async_copy(
        bufs[0], out_hbm.at[pl.ds(0, R), pl.ds(c0, COLS_W)], sems[0]
    ).wait()
    pltpu.make_async_copy(
        bufs[1].at[pl.ds(0, R_LAST)],
        out_hbm.at[pl.ds(0, R_LAST), pl.ds(c0, COLS_W)],
        sems[1],
    ).wait()


def kernel(indices):
    k = pl.kernel(
        _sc_body,
        out_type=jax.ShapeDtypeStruct((NUM_CLASSES, BATCH), jnp.float32),
        mesh=plsc.VectorSubcoreMesh(
            core_axis_name="c", subcore_axis_name="s",
            num_cores=NC, num_subcores=NS,
        ),
        scratch_types=[
            pltpu.VMEM((COLS_W,), jnp.int32),
            pltpu.VMEM((R, COLS_W), jnp.float32),
            pltpu.VMEM((R, COLS_W), jnp.float32),
            pltpu.SemaphoreType.DMA,
            pltpu.SemaphoreType.DMA,
        ],
        compiler_params=pltpu.CompilerParams(
            needs_layout_passes=False, use_tc_tiling_on_sc=True
        ),
    )
    return k(indices.astype(jnp.int32)).T


# speedup vs baseline: 3.7679x; 1.0082x over previous
import jax
import jax.numpy as jnp
from jax import lax
from jax.experimental import pallas as pl
from jax.experimental.pallas import tpu as pltpu
from jax.experimental.pallas import tpu_sc as plsc

NUM_CLASSES = 1000
BATCH = 16384
NC, NS, L = 2, 16, 16
NW = NC * NS
COLS_W = BATCH // NW
R = 64
NCH = -(-NUM_CLASSES // R)
R_LAST = NUM_CLASSES - (NCH - 1) * R
NVREG = COLS_W // L


def _sc_body(idx_hbm, out_hbm, idx_v, buf0, buf1, sem0, sem1):
    wid = lax.axis_index("s") * NC + lax.axis_index("c")
    c0 = wid * COLS_W
    pltpu.sync_copy(idx_hbm.at[pl.ds(c0, COLS_W)], idx_v)

    bufs = (buf0, buf1)
    sems = (sem0, sem1)
    zeros16 = jnp.zeros((L,), jnp.float32)
    ones16 = jnp.ones((L,), jnp.float32)
    lane_iota = lax.iota(jnp.int32, L)

    def zero_buf(buf):
        def zbody(r, _):
            for c in range(0, COLS_W, L):
                buf[r, pl.ds(c, L)] = zeros16
            return 0

        lax.fori_loop(0, R, zbody, 0)

    def in_range(row, n):
        return plsc.bitcast(row, jnp.uint32) < jnp.uint32(n)

    def process_chunk(i, b, rows_new, first):
        buf = bufs[b]
        rbase_new = i * R

        if not first:
            pltpu.make_async_copy(
                buf, out_hbm.at[pl.ds(0, R), pl.ds(c0, COLS_W)], sems[b]
            ).wait()

        for v in range(NVREG):
            idxv = idx_v[pl.ds(v * L, L)]
            col = lane_iota + (v * L)
            new_row = idxv - rbase_new
            old_row = new_row + 2 * R
            plsc.store_scatter(
                buf, [old_row, col], zeros16, mask=in_range(old_row, R)
            )
            plsc.store_scatter(
                buf, [new_row, col], ones16, mask=in_range(new_row, rows_new)
            )

        if rows_new == R:
            src, dst = buf, out_hbm.at[pl.ds(rbase_new, R), pl.ds(c0, COLS_W)]
        else:
            src = buf.at[pl.ds(0, rows_new)]
            dst = out_hbm.at[pl.ds(rbase_new, rows_new), pl.ds(c0, COLS_W)]
        pltpu.async_copy(src, dst, sems[b])

    zero_buf(buf0)
    process_chunk(0, 0, R, True)
    zero_buf(buf1)
    process_chunk(1, 1, R, True)

    def body(j, _):
        for b in range(2):
            process_chunk(j * 2 + b, b, R, False)
        return 0

    lax.fori_loop(1, (NCH - 2) // 2, body, 0)
    process_chunk(NCH - 2, 0, R, False)
    process_chunk(NCH - 1, 1, R_LAST, False)

    pltpu.make_async_copy(
        bufs[0], out_hbm.at[pl.ds(0, R), pl.ds(c0, COLS_W)], sems[0]
    ).wait()
    pltpu.make_async_copy(
        bufs[1].at[pl.ds(0, R_LAST)],
        out_hbm.at[pl.ds(0, R_LAST), pl.ds(c0, COLS_W)],
        sems[1],
    ).wait()


def kernel(indices):
    k = pl.kernel(
        _sc_body,
        out_type=jax.ShapeDtypeStruct((NUM_CLASSES, BATCH), jnp.float32),
        mesh=plsc.VectorSubcoreMesh(
            core_axis_name="c", subcore_axis_name="s",
            num_cores=NC, num_subcores=NS,
        ),
        scratch_types=[
            pltpu.VMEM((COLS_W,), jnp.int32),
            pltpu.VMEM((R, COLS_W), jnp.float32),
            pltpu.VMEM((R, COLS_W), jnp.float32),
            pltpu.SemaphoreType.DMA,
            pltpu.SemaphoreType.DMA,
        ],
        compiler_params=pltpu.CompilerParams(
            needs_layout_passes=False,
            use_tc_tiling_on_sc=True,
            disable_bounds_checks=True,
            disable_semaphore_checks=True,
            skip_device_barrier=True,
        ),
    )
    return k(indices.astype(jnp.int32)).T
